# baseline (speedup 1.0000x reference)
import jax
import jax.numpy as jnp
from jax.experimental import pallas as pl
from jax.experimental.pallas import tpu as pltpu

_K = 8192
_D = 32
_N = 8192
_BN = 256
_NB = _N // _BN


def _vq_body(z_ref, cb_ref, w_ref, b_ref,
             idx_ref, q_ref, loss_ref, perp_ref,
             pcb_ref, c2_ref, counts_ref):
    i = pl.program_id(0)
    nb = pl.num_programs(0)

    @pl.when(i == 0)
    def _init():
        pcb = jax.lax.dot_general(
            cb_ref[...].astype(jnp.bfloat16), w_ref[...].astype(jnp.bfloat16),
            (((1,), (1,)), ((), ())),
            preferred_element_type=jnp.float32) + b_ref[...]
        pcb_ref[...] = pcb
        c2_ref[...] = jnp.sum(pcb * pcb, axis=1).reshape(1, _K)
        counts_ref[...] = jnp.zeros((1, _K), jnp.float32)
        loss_ref[0, 0] = 0.0

    zb = z_ref[...]
    pcb = pcb_ref[...]
    dots = jax.lax.dot_general(
        zb.astype(jnp.bfloat16), pcb.astype(jnp.bfloat16),
        (((1,), (1,)), ((), ())),
        preferred_element_type=jnp.float32)
    z2 = jnp.sum(zb * zb, axis=1, keepdims=True)
    score = (z2 + c2_ref[...]) - 2.0 * dots
    idx = jnp.argmin(score, axis=1).astype(jnp.int32)
    idx_ref[0, 0, :] = idx

    onehot = (jax.lax.broadcasted_iota(jnp.int32, (_BN, _K), 1)
              == idx[:, None]).astype(jnp.float32)
    qb = jax.lax.dot_general(
        onehot, pcb, (((1,), (0,)), ((), ())),
        preferred_element_type=jnp.float32,
        precision=jax.lax.Precision.HIGHEST)
    q_ref[...] = qb
    counts_ref[...] += jnp.sum(onehot, axis=0).reshape(1, _K)
    loss_ref[0, 0] += jnp.sum((zb - qb) * (zb - qb))

    @pl.when(i == nb - 1)
    def _fini():
        loss_ref[0, 0] = loss_ref[0, 0] / float(_N * _D)
        probs = counts_ref[...] / float(_N)
        ent = jnp.sum(probs * jnp.log(jnp.clip(probs, 1e-10, None)))
        perp_ref[0, 0] = jnp.exp(-ent)


def kernel(z, codebook, W_proj, b_proj):
    B, C, H, W = z.shape
    flat_z = jnp.transpose(z, (0, 2, 3, 1)).reshape(-1, C)

    idx3, q, loss, perp = pl.pallas_call(
        _vq_body,
        grid=(_NB,),
        in_specs=[
            pl.BlockSpec((_BN, _D), lambda i: (i, 0)),
            pl.BlockSpec((_K, _D), lambda i: (0, 0)),
            pl.BlockSpec((_D, _D), lambda i: (0, 0)),
            pl.BlockSpec((1, _D), lambda i: (0, 0)),
        ],
        out_specs=[
            pl.BlockSpec((1, 1, _BN), lambda i: (i, 0, 0)),
            pl.BlockSpec((_BN, _D), lambda i: (i, 0)),
            pl.BlockSpec(memory_space=pltpu.SMEM),
            pl.BlockSpec(memory_space=pltpu.SMEM),
        ],
        out_shape=[
            jax.ShapeDtypeStruct((_NB, 1, _BN), jnp.int32),
            jax.ShapeDtypeStruct((_N, _D), jnp.float32),
            jax.ShapeDtypeStruct((1, 1), jnp.float32),
            jax.ShapeDtypeStruct((1, 1), jnp.float32),
        ],
        scratch_shapes=[
            pltpu.VMEM((_K, _D), jnp.float32),
            pltpu.VMEM((1, _K), jnp.float32),
            pltpu.VMEM((1, _K), jnp.float32),
        ],
    )(flat_z, codebook, W_proj, b_proj.reshape(1, _D))

    indices = idx3.reshape(_N)
    quantized = jnp.transpose(q.reshape(B, H, W, C), (0, 3, 1, 2))
    quantized = z + jax.lax.stop_gradient(quantized - z)
    loss = loss.reshape(())
    perp = perp.reshape(())
    return (quantized, indices, perp, loss, loss)
